# baseline (speedup 1.0000x reference)
import functools

import jax
import jax.numpy as jnp
from jax import lax
from jax.experimental import pallas as pl
from jax.experimental.pallas import tpu as pltpu
from jax.experimental.pallas import tpu_sc as plsc

_LANES = 16


def kernel(x, emb_table, pos_table):
    B, L = x.shape
    H = emb_table.shape[1]
    N = B * L
    NP = N // 2
    LP = L * L
    H2 = 2 * H

    info = plsc.get_sparse_core_info()
    NC, NS = info.num_cores, info.num_subcores
    NW = NC * NS
    C = 128
    K = 2
    per_w = NP // NW
    G = per_w // C
    rows_per_tile = LP // NS
    BCH = 125
    n_bch = rows_per_tile // BCH
    assert per_w * NW == NP and G * C == per_w and G % K == 0
    assert rows_per_tile * NS == LP and n_bch * BCH == rows_per_tile

    x1 = x.reshape(N)

    mesh = plsc.VectorSubcoreMesh(core_axis_name="c", subcore_axis_name="s")

    @functools.partial(
        pl.kernel,
        mesh=mesh,
        compiler_params=pltpu.CompilerParams(
            use_tc_tiling_on_sc=False, needs_layout_passes=False
        ),
        out_type=[
            jax.ShapeDtypeStruct((NP, H2), jnp.float32),
            jax.ShapeDtypeStruct((NC * LP, H2), jnp.float32),
        ],
        scratch_types=[
            pltpu.VMEM((L, H), jnp.float32),
            pltpu.VMEM((L, H), jnp.float32),
            pltpu.VMEM((per_w * 2,), jnp.int32),
            pltpu.VMEM((per_w,), jnp.int32),
            pltpu.VMEM((K, C, H2), jnp.float32),
            pltpu.SemaphoreType.DMA,
            pltpu.SemaphoreType.DMA,
        ],
    )
    def emb_lookup(x_hbm, emb_hbm, pos_hbm, out_hbm, pair_hbm,
                   emb_v, comb_v, idx_v, pidx_v, rows_v, gsem, wsem):
        cid = lax.axis_index("c")
        sid = lax.axis_index("s")
        wid = sid * NC + cid

        pltpu.sync_copy(emb_hbm.at[pl.ds(0, L)], emb_v)
        pltpu.sync_copy(pos_hbm, comb_v)

        def add_row(r, carry):
            for cg in range(H // _LANES):
                sl = pl.ds(cg * _LANES, _LANES)
                comb_v[r, sl] = comb_v[r, sl] + emb_v[r, sl]
            return carry

        lax.fori_loop(0, L, add_row, 0)

        tile_row0 = cid * LP + sid * rows_per_tile

        def build_chunk(c, carry):
            def build_row(r, carry2):
                sr = sid * rows_per_tile + c * BCH + r
                a = sr // L
                b = sr - a * L
                for cg in range(H // _LANES):
                    sl = pl.ds(cg * _LANES, _LANES)
                    rows_v[0, r, sl] = comb_v[a, sl]
                    sl2 = pl.ds(H + cg * _LANES, _LANES)
                    rows_v[0, r, sl2] = comb_v[b, sl]
                return carry2

            lax.fori_loop(0, BCH, build_row, 0)
            pltpu.sync_copy(
                rows_v.at[0].at[pl.ds(0, BCH)],
                pair_hbm.at[pl.ds(tile_row0 + c * BCH, BCH)],
            )
            return carry

        lax.fori_loop(0, n_bch, build_chunk, 0)

        pltpu.sync_copy(x_hbm.at[pl.ds(wid * per_w * 2, per_w * 2)], idx_v)
        lanes2 = lax.iota(jnp.int32, _LANES) * 2
        pbase = cid * LP

        def pair_group(j, carry):
            base = j * (2 * _LANES)
            ev = plsc.load_gather(idx_v, [lanes2 + base])
            od = plsc.load_gather(idx_v, [lanes2 + (base + 1)])
            pidx_v[pl.ds(j * _LANES, _LANES)] = ev * L + od + pbase
            return carry

        lax.fori_loop(0, per_w // _LANES, pair_group, 0)

        plsc.subcore_barrier()

        def wave(w, carry):
            g0 = w * K
            gd = [
                pltpu.async_copy(
                    pair_hbm.at[pidx_v.at[pl.ds((g0 + b) * C, C)]],
                    rows_v.at[b],
                    gsem,
                )
                for b in range(K)
            ]
            for d in gd:
                d.wait()
            wd = [
                pltpu.async_copy(
                    rows_v.at[b],
                    out_hbm.at[pl.ds(wid * per_w + (g0 + b) * C, C)],
                    wsem,
                )
                for b in range(K)
            ]
            for d in wd:
                d.wait()
            return carry

        lax.fori_loop(0, G // K, wave, 0)

    out, _ = emb_lookup(x1, emb_table, pos_table)
    return out.reshape(B, L, H)


# speedup vs baseline: 1.7438x; 1.7438x over previous
import functools

import jax
import jax.numpy as jnp
from jax import lax
from jax.experimental import pallas as pl
from jax.experimental.pallas import tpu as pltpu
from jax.experimental.pallas import tpu_sc as plsc

_LANES = 16


def kernel(x, emb_table, pos_table):
    B, L = x.shape
    H = emb_table.shape[1]
    N = B * L
    NP = N // 2
    LP = L * L
    H2 = 2 * H

    info = plsc.get_sparse_core_info()
    NC, NS = info.num_cores, info.num_subcores
    NW = NC * NS
    C = 128
    K = 2
    per_w = NP // NW
    G = per_w // C
    rows_per_tile = LP // NS
    BCH = 125
    n_bch = rows_per_tile // BCH
    assert per_w * NW == NP and G * C == per_w and G % K == 0
    assert rows_per_tile * NS == LP and n_bch * BCH == rows_per_tile

    x1 = x.reshape(N)
    emb_s = lax.slice(emb_table, (0, 0), (L, H))

    mesh = plsc.VectorSubcoreMesh(core_axis_name="c", subcore_axis_name="s")

    @functools.partial(
        pl.kernel,
        mesh=mesh,
        compiler_params=pltpu.CompilerParams(
            use_tc_tiling_on_sc=False, needs_layout_passes=False
        ),
        out_type=[
            jax.ShapeDtypeStruct((NP, H2), jnp.float32),
            jax.ShapeDtypeStruct((NC * LP, H2), jnp.float32),
        ],
        scratch_types=[
            pltpu.VMEM((L, H), jnp.float32),
            pltpu.VMEM((L, H), jnp.float32),
            pltpu.VMEM((per_w * 2,), jnp.int32),
            pltpu.VMEM((per_w,), jnp.int32),
            pltpu.VMEM((K, C, H2), jnp.float32),
            pltpu.SemaphoreType.DMA,
            pltpu.SemaphoreType.DMA,
        ],
    )
    def emb_lookup(x_hbm, emb_hbm, pos_hbm, out_hbm, pair_hbm,
                   emb_v, comb_v, idx_v, pidx_v, rows_v, gsem, wsem):
        cid = lax.axis_index("c")
        sid = lax.axis_index("s")
        wid = sid * NC + cid

        pltpu.sync_copy(emb_hbm, emb_v)
        pltpu.sync_copy(pos_hbm, comb_v)

        def add_row(r, carry):
            for cg in range(H // _LANES):
                sl = pl.ds(cg * _LANES, _LANES)
                comb_v[r, sl] = comb_v[r, sl] + emb_v[r, sl]
            return carry

        lax.fori_loop(0, L, add_row, 0)

        tile_row0 = cid * LP + sid * rows_per_tile

        def build_chunk(c, carry):
            def build_row(r, carry2):
                sr = sid * rows_per_tile + c * BCH + r
                a = sr // L
                b = sr - a * L
                for cg in range(H // _LANES):
                    sl = pl.ds(cg * _LANES, _LANES)
                    rows_v[0, r, sl] = comb_v[a, sl]
                    sl2 = pl.ds(H + cg * _LANES, _LANES)
                    rows_v[0, r, sl2] = comb_v[b, sl]
                return carry2

            lax.fori_loop(0, BCH, build_row, 0)
            pltpu.sync_copy(
                rows_v.at[0].at[pl.ds(0, BCH)],
                pair_hbm.at[pl.ds(tile_row0 + c * BCH, BCH)],
            )
            return carry

        lax.fori_loop(0, n_bch, build_chunk, 0)

        pltpu.sync_copy(x_hbm.at[pl.ds(wid * per_w * 2, per_w * 2)], idx_v)
        lanes2 = lax.iota(jnp.int32, _LANES) * 2
        pbase = cid * LP

        def pair_group(j, carry):
            base = j * (2 * _LANES)
            ev = plsc.load_gather(idx_v, [lanes2 + base])
            od = plsc.load_gather(idx_v, [lanes2 + (base + 1)])
            pidx_v[pl.ds(j * _LANES, _LANES)] = ev * L + od + pbase
            return carry

        lax.fori_loop(0, per_w // _LANES, pair_group, 0)

        plsc.subcore_barrier()

        def wave(w, carry):
            g0 = w * K
            gd = [
                pltpu.async_copy(
                    pair_hbm.at[pidx_v.at[pl.ds((g0 + b) * C, C)]],
                    rows_v.at[b],
                    gsem,
                )
                for b in range(K)
            ]
            for d in gd:
                d.wait()
            wd = [
                pltpu.async_copy(
                    rows_v.at[b],
                    out_hbm.at[pl.ds(wid * per_w + (g0 + b) * C, C)],
                    wsem,
                )
                for b in range(K)
            ]
            for d in wd:
                d.wait()
            return carry

        lax.fori_loop(0, G // K, wave, 0)

    out, _ = emb_lookup(x1, emb_s, pos_table)
    return out.reshape(B, L, H)
